# baseline (speedup 1.0000x reference)
import functools

import jax
import jax.numpy as jnp
from jax import lax
from jax.experimental import pallas as pl
from jax.experimental.pallas import tpu as pltpu
from jax.experimental.pallas import tpu_sc as plsc

D = 32
MIN_RATING = 0.5
MAX_RATING = 5.0

NC = 2
NS = 16
NW = NC * NS
CH = 64
CHUNK = 128


def _emb_body(uids_hbm, iids_hbm, uemb2, iemb2, ue_out, ie_out,
              ids_v, ridx_v, buf, outc, sem):
    b_per_w = ids_v.shape[1]
    wid = lax.axis_index("s") * NC + lax.axis_index("c")
    base = wid * b_per_w

    pltpu.sync_copy(uids_hbm.at[pl.ds(base, b_per_w)], ids_v.at[0])
    pltpu.sync_copy(iids_hbm.at[pl.ds(base, b_per_w)], ids_v.at[1])

    for t, (tab, out_ref) in enumerate(((uemb2, ue_out), (iemb2, ie_out))):
        t_vec = jnp.full((16,), t, dtype=jnp.int32)

        def chunk_body(g, carry, t=t, t_vec=t_vec, tab=tab, out_ref=out_ref):
            cbase = g * CH
            for q in range(CH // 16):
                ids16 = ids_v[t, pl.ds(cbase + q * 16, 16)]
                ridx_v[pl.ds(q * 16, 16)] = ids16 >> 3
            pltpu.async_copy(tab.at[ridx_v], buf, sem).wait()
            for gg in range(CH * D // 16):
                j = jnp.arange(gg * 16, gg * 16 + 16, dtype=jnp.int32)
                i0 = j >> 5
                i2 = j & 31
                ids16 = plsc.load_gather(ids_v, [t_vec, cbase + i0])
                col = ((ids16 & 7) << 5) + i2
                vals = plsc.load_gather(buf, [i0, col])
                outc[gg // 2, pl.ds((gg % 2) * 16, 16)] = vals
            pltpu.sync_copy(outc, out_ref.at[pl.ds(base + cbase, CH)])
            return carry

        lax.fori_loop(0, b_per_w // CH, chunk_body, None)


@functools.partial(jax.jit, static_argnames=("batch",))
def _sc_emb(uids, iids, uemb3, iemb3, *, batch):
    b_per_w = batch // NW
    mesh = plsc.VectorSubcoreMesh(core_axis_name="c", subcore_axis_name="s")
    f = pl.kernel(
        _emb_body,
        out_type=[
            jax.ShapeDtypeStruct((batch, D), jnp.float32),
            jax.ShapeDtypeStruct((batch, D), jnp.float32),
        ],
        mesh=mesh,
        compiler_params=pltpu.CompilerParams(needs_layout_passes=False),
        scratch_types=[
            pltpu.VMEM((2, b_per_w), jnp.int32),
            pltpu.VMEM((CH,), jnp.int32),
            pltpu.VMEM((CH, 8 * D), jnp.float32),
            pltpu.VMEM((CH, D), jnp.float32),
            pltpu.SemaphoreType.DMA,
        ],
    )
    return f(uids, iids, uemb3, iemb3)


def _bias_body(uids_hbm, iids_hbm, ubias_hbm, ibias_hbm, ub_out, ib_out,
               uidx, iidx, ubv, ibv, sem):
    n_chunks = uidx.shape[0]
    b_per_w = n_chunks * CHUNK
    wid = lax.axis_index("s") * NC + lax.axis_index("c")
    base = wid * b_per_w

    pltpu.sync_copy(uids_hbm.at[pl.ds(wid * n_chunks, n_chunks)], uidx)
    pltpu.sync_copy(iids_hbm.at[pl.ds(wid * n_chunks, n_chunks)], iidx)

    copies = []
    for j in range(n_chunks):
        sl = pl.ds(j * CHUNK, CHUNK)
        copies.append(pltpu.async_copy(ubias_hbm.at[uidx.at[j]], ubv.at[sl], sem))
        copies.append(pltpu.async_copy(ibias_hbm.at[iidx.at[j]], ibv.at[sl], sem))
    for c in copies:
        c.wait()

    out_sl = pl.ds(base, b_per_w)
    pltpu.sync_copy(ubv, ub_out.at[out_sl])
    pltpu.sync_copy(ibv, ib_out.at[out_sl])


@functools.partial(jax.jit, static_argnames=("batch",))
def _sc_bias(uids2, iids2, ubias, ibias, *, batch):
    b_per_w = batch // NW
    n_chunks = b_per_w // CHUNK
    mesh = plsc.VectorSubcoreMesh(core_axis_name="c", subcore_axis_name="s")
    f = pl.kernel(
        _bias_body,
        out_type=[
            jax.ShapeDtypeStruct((batch,), jnp.float32),
            jax.ShapeDtypeStruct((batch,), jnp.float32),
        ],
        mesh=mesh,
        compiler_params=pltpu.CompilerParams(use_tc_tiling_on_sc=False),
        scratch_types=[
            pltpu.VMEM((n_chunks, CHUNK), jnp.int32),
            pltpu.VMEM((n_chunks, CHUNK), jnp.int32),
            pltpu.VMEM((b_per_w,), jnp.float32),
            pltpu.VMEM((b_per_w,), jnp.float32),
            pltpu.SemaphoreType.DMA,
        ],
    )
    return f(uids2, iids2, ubias, ibias)


def _mlp_body(ue_ref, ie_ref, ub_ref, ib_ref, w1a_ref, w1b_ref, b1_ref,
              w2_ref, b2_ref, w3_ref, b3_ref, out_ref):
    h = ue_ref[...] @ w1a_ref[...] + ie_ref[...] @ w1b_ref[...] + b1_ref[...]
    h = jnp.maximum(h, 0.0)
    h = jnp.maximum(h @ w2_ref[...] + b2_ref[...], 0.0)
    p = h @ w3_ref[...] + b3_ref[...] + ub_ref[...] + ib_ref[...]
    out_ref[...] = jnp.clip(p, MIN_RATING, MAX_RATING)


@functools.partial(jax.jit, static_argnames=("batch",))
def _tc_mlp(ue, ie, ub, ib, w1a, w1b, b1, w2, b2, w3, b3, *, batch):
    blk = 2048
    grid = (batch // blk,)
    full = lambda shape: pl.BlockSpec(shape, lambda i: (0, 0))
    return pl.pallas_call(
        _mlp_body,
        grid=grid,
        in_specs=[
            pl.BlockSpec((blk, D), lambda i: (i, 0)),
            pl.BlockSpec((blk, D), lambda i: (i, 0)),
            pl.BlockSpec((blk, 1), lambda i: (i, 0)),
            pl.BlockSpec((blk, 1), lambda i: (i, 0)),
            full((D, 32)),
            full((D, 32)),
            full((1, 32)),
            full((32, 16)),
            full((1, 16)),
            full((16, 1)),
            full((1, 1)),
        ],
        out_specs=pl.BlockSpec((blk, 1), lambda i: (i, 0)),
        out_shape=jax.ShapeDtypeStruct((batch, 1), jnp.float32),
    )(ue, ie, ub, ib, w1a, w1b, b1, w2, b2, w3, b3)


def kernel(user_ids, item_ids, user_emb, item_emb, user_bias, item_bias,
           W1, b1, W2, b2, W3, b3):
    batch = user_ids.shape[0]
    uids = user_ids.astype(jnp.int32)
    iids = item_ids.astype(jnp.int32)
    ue, ie = _sc_emb(
        uids, iids,
        user_emb.reshape(-1, 8 * D), item_emb.reshape(-1, 8 * D), batch=batch)
    ub, ib = _sc_bias(
        uids.reshape(batch // CHUNK, CHUNK), iids.reshape(batch // CHUNK, CHUNK),
        user_bias.sum(axis=1), item_bias.sum(axis=1), batch=batch)
    return _tc_mlp(
        ue, ie, ub.reshape(batch, 1), ib.reshape(batch, 1),
        W1[:D], W1[D:], b1.reshape(1, -1), W2, b2.reshape(1, -1),
        W3, b3.reshape(1, 1), batch=batch)


# speedup vs baseline: 2.1559x; 2.1559x over previous
import functools

import jax
import jax.numpy as jnp
from jax import lax
from jax.experimental import pallas as pl
from jax.experimental.pallas import tpu as pltpu
from jax.experimental.pallas import tpu_sc as plsc

D = 32
MIN_RATING = 0.5
MAX_RATING = 5.0

NC = 2
NS = 16
NW = NC * NS
CH = 64
CHUNK = 128


def _emb_body(uids_hbm, iids_hbm, uemb2, iemb2, ue_out, ie_out,
              ids_v, ridx_v, buf, outc, sem):
    b_per_w = ids_v.shape[1]
    wid = lax.axis_index("s") * NC + lax.axis_index("c")
    base = wid * b_per_w

    pltpu.sync_copy(uids_hbm.at[pl.ds(base, b_per_w)], ids_v.at[0])
    pltpu.sync_copy(iids_hbm.at[pl.ds(base, b_per_w)], ids_v.at[1])

    for t, (tab, out_ref) in enumerate(((uemb2, ue_out), (iemb2, ie_out))):
        t_vec = jnp.full((16,), t, dtype=jnp.int32)

        iota16 = jnp.arange(16, dtype=jnp.int32)

        def chunk_body(g, carry, t=t, t_vec=t_vec, tab=tab, out_ref=out_ref):
            cbase = g * CH
            for q in range(CH // 16):
                ids16 = ids_v[t, pl.ds(cbase + q * 16, 16)]
                rows16 = ids16 >> 3
                for k in range(16):
                    rk = jnp.sum(jnp.where(iota16 == k, rows16, 0))
                    pltpu.async_copy(tab.at[rk], buf.at[q * 16 + k], sem)
            pltpu.make_async_copy(tab.at[pl.ds(0, CH)], buf, sem).wait()
            for gg in range(CH * D // 16):
                j = jnp.arange(gg * 16, gg * 16 + 16, dtype=jnp.int32)
                i0 = j >> 5
                i2 = j & 31
                ids16 = plsc.load_gather(ids_v, [t_vec, cbase + i0])
                i1 = ids16 & 7
                vals = plsc.load_gather(buf, [i0, i1, i2])
                outc[gg // 2, pl.ds((gg % 2) * 16, 16)] = vals
            pltpu.sync_copy(outc, out_ref.at[pl.ds(base + cbase, CH)])
            return carry

        lax.fori_loop(0, b_per_w // CH, chunk_body, None)


@functools.partial(jax.jit, static_argnames=("batch",))
def _sc_emb(uids, iids, uemb3, iemb3, *, batch):
    b_per_w = batch // NW
    mesh = plsc.VectorSubcoreMesh(core_axis_name="c", subcore_axis_name="s")
    f = pl.kernel(
        _emb_body,
        out_type=[
            jax.ShapeDtypeStruct((batch, D), jnp.float32),
            jax.ShapeDtypeStruct((batch, D), jnp.float32),
        ],
        mesh=mesh,
        compiler_params=pltpu.CompilerParams(needs_layout_passes=False),
        scratch_types=[
            pltpu.VMEM((2, b_per_w), jnp.int32),
            pltpu.VMEM((CH,), jnp.int32),
            pltpu.VMEM((CH, 8, D), jnp.float32),
            pltpu.VMEM((CH, D), jnp.float32),
            pltpu.SemaphoreType.DMA,
        ],
    )
    return f(uids, iids, uemb3, iemb3)


def _bias_body(uids_hbm, iids_hbm, ubias_hbm, ibias_hbm, ub_out, ib_out,
               uidx, iidx, ubv, ibv, sem):
    n_chunks = uidx.shape[0]
    b_per_w = n_chunks * CHUNK
    wid = lax.axis_index("s") * NC + lax.axis_index("c")
    base = wid * b_per_w

    pltpu.sync_copy(uids_hbm.at[pl.ds(wid * n_chunks, n_chunks)], uidx)
    pltpu.sync_copy(iids_hbm.at[pl.ds(wid * n_chunks, n_chunks)], iidx)

    copies = []
    for j in range(n_chunks):
        sl = pl.ds(j * CHUNK, CHUNK)
        copies.append(pltpu.async_copy(ubias_hbm.at[uidx.at[j]], ubv.at[sl], sem))
        copies.append(pltpu.async_copy(ibias_hbm.at[iidx.at[j]], ibv.at[sl], sem))
    for c in copies:
        c.wait()

    out_sl = pl.ds(base, b_per_w)
    pltpu.sync_copy(ubv, ub_out.at[out_sl])
    pltpu.sync_copy(ibv, ib_out.at[out_sl])


@functools.partial(jax.jit, static_argnames=("batch",))
def _sc_bias(uids2, iids2, ubias, ibias, *, batch):
    b_per_w = batch // NW
    n_chunks = b_per_w // CHUNK
    mesh = plsc.VectorSubcoreMesh(core_axis_name="c", subcore_axis_name="s")
    f = pl.kernel(
        _bias_body,
        out_type=[
            jax.ShapeDtypeStruct((batch,), jnp.float32),
            jax.ShapeDtypeStruct((batch,), jnp.float32),
        ],
        mesh=mesh,
        compiler_params=pltpu.CompilerParams(use_tc_tiling_on_sc=False),
        scratch_types=[
            pltpu.VMEM((n_chunks, CHUNK), jnp.int32),
            pltpu.VMEM((n_chunks, CHUNK), jnp.int32),
            pltpu.VMEM((b_per_w,), jnp.float32),
            pltpu.VMEM((b_per_w,), jnp.float32),
            pltpu.SemaphoreType.DMA,
        ],
    )
    return f(uids2, iids2, ubias, ibias)


def _mlp_body(ue_ref, ie_ref, ub_ref, ib_ref, w1a_ref, w1b_ref, b1_ref,
              w2_ref, b2_ref, w3_ref, b3_ref, out_ref):
    h = ue_ref[...] @ w1a_ref[...] + ie_ref[...] @ w1b_ref[...] + b1_ref[...]
    h = jnp.maximum(h, 0.0)
    h = jnp.maximum(h @ w2_ref[...] + b2_ref[...], 0.0)
    p = h @ w3_ref[...] + b3_ref[...] + ub_ref[...] + ib_ref[...]
    out_ref[...] = jnp.clip(p, MIN_RATING, MAX_RATING)


@functools.partial(jax.jit, static_argnames=("batch",))
def _tc_mlp(ue, ie, ub, ib, w1a, w1b, b1, w2, b2, w3, b3, *, batch):
    blk = 2048
    grid = (batch // blk,)
    full = lambda shape: pl.BlockSpec(shape, lambda i: (0, 0))
    return pl.pallas_call(
        _mlp_body,
        grid=grid,
        in_specs=[
            pl.BlockSpec((blk, D), lambda i: (i, 0)),
            pl.BlockSpec((blk, D), lambda i: (i, 0)),
            pl.BlockSpec((blk, 1), lambda i: (i, 0)),
            pl.BlockSpec((blk, 1), lambda i: (i, 0)),
            full((D, 32)),
            full((D, 32)),
            full((1, 32)),
            full((32, 16)),
            full((1, 16)),
            full((16, 1)),
            full((1, 1)),
        ],
        out_specs=pl.BlockSpec((blk, 1), lambda i: (i, 0)),
        out_shape=jax.ShapeDtypeStruct((batch, 1), jnp.float32),
    )(ue, ie, ub, ib, w1a, w1b, b1, w2, b2, w3, b3)


def kernel(user_ids, item_ids, user_emb, item_emb, user_bias, item_bias,
           W1, b1, W2, b2, W3, b3):
    batch = user_ids.shape[0]
    uids = user_ids.astype(jnp.int32)
    iids = item_ids.astype(jnp.int32)
    ue, ie = _sc_emb(
        uids, iids,
        user_emb.reshape(-1, 8, D), item_emb.reshape(-1, 8, D), batch=batch)
    ub, ib = _sc_bias(
        uids.reshape(batch // CHUNK, CHUNK), iids.reshape(batch // CHUNK, CHUNK),
        user_bias.sum(axis=1), item_bias.sum(axis=1), batch=batch)
    return _tc_mlp(
        ue, ie, ub.reshape(batch, 1), ib.reshape(batch, 1),
        W1[:D], W1[D:], b1.reshape(1, -1), W2, b2.reshape(1, -1),
        W3, b3.reshape(1, 1), batch=batch)


# speedup vs baseline: 2.2775x; 1.0564x over previous
import functools

import jax
import jax.numpy as jnp
from jax import lax
from jax.experimental import pallas as pl
from jax.experimental.pallas import tpu as pltpu
from jax.experimental.pallas import tpu_sc as plsc

D = 32
MIN_RATING = 0.5
MAX_RATING = 5.0

NC = 2
NS = 16
NW = NC * NS
CH = 32
CHUNK = 128


def _emb_body(uids_hbm, iids_hbm, uemb2, iemb2, ue_out, ie_out,
              ids_v, buf, outc, sem0, sem1):
    b_per_w = ids_v.shape[1]
    n_chunks = b_per_w // CH
    wid = lax.axis_index("s") * NC + lax.axis_index("c")
    base = wid * b_per_w

    pltpu.sync_copy(uids_hbm.at[pl.ds(base, b_per_w)], ids_v.at[0])
    pltpu.sync_copy(iids_hbm.at[pl.ds(base, b_per_w)], ids_v.at[1])

    iota16 = jnp.arange(16, dtype=jnp.int32)

    for t, (tab, out_ref) in enumerate(((uemb2, ue_out), (iemb2, ie_out))):
        t_vec = jnp.full((16,), t, dtype=jnp.int32)

        def fire(g, t=t, tab=tab):
            cbase = g * CH
            half = (g & 1) * CH

            def go(sem):
                for q in range(CH // 16):
                    ids16 = ids_v[t, pl.ds(cbase + q * 16, 16)]
                    rows16 = ids16 >> 3
                    for k in range(16):
                        rk = jnp.sum(jnp.where(iota16 == k, rows16, 0))
                        pltpu.async_copy(tab.at[rk], buf.at[half + q * 16 + k], sem)

            @pl.when((g & 1) == 0)
            def _():
                go(sem0)

            @pl.when((g & 1) == 1)
            def _():
                go(sem1)

        def drain(g, tab=tab):
            half = (g & 1) * CH
            dst = buf.at[pl.ds(half, CH)]

            @pl.when(g & 1 == 0)
            def _():
                pltpu.make_async_copy(tab.at[pl.ds(0, CH)], dst, sem0).wait()

            @pl.when(g & 1 == 1)
            def _():
                pltpu.make_async_copy(tab.at[pl.ds(0, CH)], dst, sem1).wait()

        fire(0)

        def chunk_body(g, carry, t=t, t_vec=t_vec, tab=tab, out_ref=out_ref,
                       fire=fire, drain=drain):
            cbase = g * CH
            half = (g & 1) * CH

            @pl.when(g + 1 < n_chunks)
            def _():
                fire(g + 1)

            drain(g)
            for gg in range(CH * D // 16):
                j = jnp.arange(gg * 16, gg * 16 + 16, dtype=jnp.int32)
                i0 = j >> 5
                i2 = j & 31
                ids16 = plsc.load_gather(ids_v, [t_vec, cbase + i0])
                i1 = ids16 & 7
                vals = plsc.load_gather(buf, [half + i0, i1, i2])
                outc[gg // 2, pl.ds((gg % 2) * 16, 16)] = vals
            pltpu.sync_copy(outc, out_ref.at[pl.ds(base + cbase, CH)])
            return carry

        lax.fori_loop(0, n_chunks, chunk_body, None)


@functools.partial(jax.jit, static_argnames=("batch",))
def _sc_emb(uids, iids, uemb3, iemb3, *, batch):
    b_per_w = batch // NW
    mesh = plsc.VectorSubcoreMesh(core_axis_name="c", subcore_axis_name="s")
    f = pl.kernel(
        _emb_body,
        out_type=[
            jax.ShapeDtypeStruct((batch, D), jnp.float32),
            jax.ShapeDtypeStruct((batch, D), jnp.float32),
        ],
        mesh=mesh,
        compiler_params=pltpu.CompilerParams(needs_layout_passes=False),
        scratch_types=[
            pltpu.VMEM((2, b_per_w), jnp.int32),
            pltpu.VMEM((2 * CH, 8, D), jnp.float32),
            pltpu.VMEM((CH, D), jnp.float32),
            pltpu.SemaphoreType.DMA,
            pltpu.SemaphoreType.DMA,
        ],
    )
    return f(uids, iids, uemb3, iemb3)


def _bias_body(uids_hbm, iids_hbm, ubias_hbm, ibias_hbm, ub_out, ib_out,
               uidx, iidx, ubv, ibv, sem):
    n_chunks = uidx.shape[0]
    b_per_w = n_chunks * CHUNK
    wid = lax.axis_index("s") * NC + lax.axis_index("c")
    base = wid * b_per_w

    pltpu.sync_copy(uids_hbm.at[pl.ds(wid * n_chunks, n_chunks)], uidx)
    pltpu.sync_copy(iids_hbm.at[pl.ds(wid * n_chunks, n_chunks)], iidx)

    copies = []
    for j in range(n_chunks):
        sl = pl.ds(j * CHUNK, CHUNK)
        copies.append(pltpu.async_copy(ubias_hbm.at[uidx.at[j]], ubv.at[sl], sem))
        copies.append(pltpu.async_copy(ibias_hbm.at[iidx.at[j]], ibv.at[sl], sem))
    for c in copies:
        c.wait()

    out_sl = pl.ds(base, b_per_w)
    pltpu.sync_copy(ubv, ub_out.at[out_sl])
    pltpu.sync_copy(ibv, ib_out.at[out_sl])


@functools.partial(jax.jit, static_argnames=("batch",))
def _sc_bias(uids2, iids2, ubias, ibias, *, batch):
    b_per_w = batch // NW
    n_chunks = b_per_w // CHUNK
    mesh = plsc.VectorSubcoreMesh(core_axis_name="c", subcore_axis_name="s")
    f = pl.kernel(
        _bias_body,
        out_type=[
            jax.ShapeDtypeStruct((batch,), jnp.float32),
            jax.ShapeDtypeStruct((batch,), jnp.float32),
        ],
        mesh=mesh,
        compiler_params=pltpu.CompilerParams(use_tc_tiling_on_sc=False),
        scratch_types=[
            pltpu.VMEM((n_chunks, CHUNK), jnp.int32),
            pltpu.VMEM((n_chunks, CHUNK), jnp.int32),
            pltpu.VMEM((b_per_w,), jnp.float32),
            pltpu.VMEM((b_per_w,), jnp.float32),
            pltpu.SemaphoreType.DMA,
        ],
    )
    return f(uids2, iids2, ubias, ibias)


def _mlp_body(ue_ref, ie_ref, ub_ref, ib_ref, w1a_ref, w1b_ref, b1_ref,
              w2_ref, b2_ref, w3_ref, b3_ref, out_ref):
    h = ue_ref[...] @ w1a_ref[...] + ie_ref[...] @ w1b_ref[...] + b1_ref[...]
    h = jnp.maximum(h, 0.0)
    h = jnp.maximum(h @ w2_ref[...] + b2_ref[...], 0.0)
    p = h @ w3_ref[...] + b3_ref[...] + ub_ref[...] + ib_ref[...]
    out_ref[...] = jnp.clip(p, MIN_RATING, MAX_RATING)


@functools.partial(jax.jit, static_argnames=("batch",))
def _tc_mlp(ue, ie, ub, ib, w1a, w1b, b1, w2, b2, w3, b3, *, batch):
    blk = 2048
    grid = (batch // blk,)
    full = lambda shape: pl.BlockSpec(shape, lambda i: (0, 0))
    return pl.pallas_call(
        _mlp_body,
        grid=grid,
        in_specs=[
            pl.BlockSpec((blk, D), lambda i: (i, 0)),
            pl.BlockSpec((blk, D), lambda i: (i, 0)),
            pl.BlockSpec((blk, 1), lambda i: (i, 0)),
            pl.BlockSpec((blk, 1), lambda i: (i, 0)),
            full((D, 32)),
            full((D, 32)),
            full((1, 32)),
            full((32, 16)),
            full((1, 16)),
            full((16, 1)),
            full((1, 1)),
        ],
        out_specs=pl.BlockSpec((blk, 1), lambda i: (i, 0)),
        out_shape=jax.ShapeDtypeStruct((batch, 1), jnp.float32),
    )(ue, ie, ub, ib, w1a, w1b, b1, w2, b2, w3, b3)


def kernel(user_ids, item_ids, user_emb, item_emb, user_bias, item_bias,
           W1, b1, W2, b2, W3, b3):
    batch = user_ids.shape[0]
    uids = user_ids.astype(jnp.int32)
    iids = item_ids.astype(jnp.int32)
    ue, ie = _sc_emb(
        uids, iids,
        user_emb.reshape(-1, 8, D), item_emb.reshape(-1, 8, D), batch=batch)
    ub, ib = _sc_bias(
        uids.reshape(batch // CHUNK, CHUNK), iids.reshape(batch // CHUNK, CHUNK),
        user_bias.sum(axis=1), item_bias.sum(axis=1), batch=batch)
    return _tc_mlp(
        ue, ie, ub.reshape(batch, 1), ib.reshape(batch, 1),
        W1[:D], W1[D:], b1.reshape(1, -1), W2, b2.reshape(1, -1),
        W3, b3.reshape(1, 1), batch=batch)
